# baseline (speedup 1.0000x reference)
import functools

import jax
import jax.numpy as jnp
from jax.experimental import pallas as pl
from jax.experimental.pallas import tpu as pltpu

NUM_EXPERTS = 16
TOP_K = 2
BT = 256


def _moe_dense_body(x_ref, wr_ref, br_ref, we_ref, be_ref, out_ref, gates_scr):
    e = pl.program_id(1)

    @pl.when(e == 0)
    def _():
        logits = (
            jnp.dot(x_ref[...], wr_ref[...], preferred_element_type=jnp.float32)
            + br_ref[...]
        )
        m1 = jnp.max(logits, axis=-1, keepdims=True)
        masked = jnp.where(logits >= m1, -jnp.inf, logits)
        m2 = jnp.max(masked, axis=-1, keepdims=True)
        denom = 1.0 + jnp.exp(m2 - m1)
        gates = jnp.where(logits >= m2, jnp.exp(logits - m1) / denom, 0.0)
        gates_scr[...] = gates
        out_ref[...] = jnp.zeros_like(out_ref)

    onehot = (
        jax.lax.broadcasted_iota(jnp.int32, (BT, NUM_EXPERTS), 1) == e
    ).astype(jnp.float32)
    gate_e = jnp.sum(gates_scr[...] * onehot, axis=-1, keepdims=True)
    y = (
        jnp.dot(x_ref[...], we_ref[0], preferred_element_type=jnp.float32)
        + be_ref[0]
    )
    out_ref[...] += gate_e * y


def kernel(x, W_router, b_router, W_experts, b_experts):
    n_tokens, d_model = x.shape
    grid = (n_tokens // BT, NUM_EXPERTS)
    return pl.pallas_call(
        _moe_dense_body,
        grid=grid,
        in_specs=[
            pl.BlockSpec((BT, d_model), lambda t, e: (t, 0)),
            pl.BlockSpec((d_model, NUM_EXPERTS), lambda t, e: (0, 0)),
            pl.BlockSpec((1, NUM_EXPERTS), lambda t, e: (0, 0)),
            pl.BlockSpec((1, d_model, d_model), lambda t, e: (e, 0, 0)),
            pl.BlockSpec((1, 1, d_model), lambda t, e: (e, 0, 0)),
        ],
        out_specs=pl.BlockSpec((BT, d_model), lambda t, e: (t, 0)),
        out_shape=jax.ShapeDtypeStruct((n_tokens, d_model), x.dtype),
        scratch_shapes=[pltpu.VMEM((BT, NUM_EXPERTS), jnp.float32)],
        compiler_params=pltpu.CompilerParams(
            dimension_semantics=("parallel", "arbitrary"),
        ),
    )(
        x,
        W_router,
        b_router.reshape(1, NUM_EXPERTS),
        W_experts,
        b_experts.reshape(NUM_EXPERTS, 1, d_model),
    )


# speedup vs baseline: 1.1882x; 1.1882x over previous
import functools

import jax
import jax.numpy as jnp
from jax import lax
from jax.experimental import pallas as pl
from jax.experimental.pallas import tpu as pltpu
from jax.experimental.pallas import tpu_sc as plsc

NUM_EXPERTS = 16
TOP_K = 2
N_TOKENS = 4096
D_MODEL = 1024
N_PAIRS = N_TOKENS * TOP_K
BB = 256
P_MAX = N_PAIRS + NUM_EXPERTS * BB
NB = P_MAX // BB
NBLK_PAD = 64
NC, NS, L = 2, 16, 16
SH_OFF = 32


def _router_body(x_ref, wr_ref, br_ref, ei_ref, g_ref):
    logits = (
        jnp.dot(x_ref[...], wr_ref[...], preferred_element_type=jnp.float32)
        + br_ref[...]
    )
    iota = jax.lax.broadcasted_iota(jnp.int32, logits.shape, 1)
    m1 = jnp.max(logits, axis=-1, keepdims=True)
    e0 = jnp.min(
        jnp.where(logits >= m1, iota, NUM_EXPERTS), axis=-1, keepdims=True
    )
    masked = jnp.where(logits >= m1, -jnp.inf, logits)
    m2 = jnp.max(masked, axis=-1, keepdims=True)
    e1 = jnp.min(
        jnp.where(masked >= m2, iota, NUM_EXPERTS), axis=-1, keepdims=True
    )
    t = jnp.exp(m2 - m1)
    g0 = 1.0 / (1.0 + t)
    ei_ref[...] = jnp.concatenate([e0, e1], axis=1)
    g_ref[...] = jnp.concatenate([g0, 1.0 - g0], axis=1)


def _router(x, W_router, b_router):
    bt = 1024
    return pl.pallas_call(
        _router_body,
        grid=(N_TOKENS // bt,),
        in_specs=[
            pl.BlockSpec((bt, D_MODEL), lambda i: (i, 0)),
            pl.BlockSpec((D_MODEL, NUM_EXPERTS), lambda i: (0, 0)),
            pl.BlockSpec((1, NUM_EXPERTS), lambda i: (0, 0)),
        ],
        out_specs=[
            pl.BlockSpec((bt, TOP_K), lambda i: (i, 0)),
            pl.BlockSpec((bt, TOP_K), lambda i: (i, 0)),
        ],
        out_shape=[
            jax.ShapeDtypeStruct((N_TOKENS, TOP_K), jnp.int32),
            jax.ShapeDtypeStruct((N_TOKENS, TOP_K), jnp.float32),
        ],
    )(x, W_router, b_router.reshape(1, NUM_EXPERTS))


def _dispatch_body(
    ep_hbm, gp_hbm, x_hbm,
    xg_hbm, gs_hbm, blk_hbm, posp_hbm,
    epv, gv, tokv, gtsv, prsv, posv, cntv, allcnt, blkv, xbuf,
    counts_sh, sem,
):
    c = lax.axis_index("c")
    s = lax.axis_index("s")
    expert = s
    iota = lax.iota(jnp.int32, L)

    pltpu.sync_copy(ep_hbm, epv)
    pltpu.sync_copy(gp_hbm, gv)

    def zbody(j, carry):
        posv[pl.ds(j * L, L)] = jnp.zeros((L,), jnp.int32)
        return carry

    lax.fori_loop(0, N_PAIRS // L, zbody, 0)

    def cbody(j, cnt):
        ev = epv[pl.ds(j * L, L)]
        mask = ev == expert
        pvec = j * L + iota
        cs = plsc.cumsum(jnp.where(mask, 1, 0))
        dest = cnt + cs - 1
        plsc.store_scatter(tokv, [dest], pvec >> 1, mask=mask)
        plsc.store_scatter(gtsv, [dest], gv[pl.ds(j * L, L)], mask=mask)
        plsc.store_scatter(prsv, [dest], pvec, mask=mask)
        return cnt + cs[L - 1]

    cnt = lax.fori_loop(0, N_PAIRS // L, cbody, jnp.int32(0))

    cntv[...] = jnp.broadcast_to(cnt, (L,))
    pltpu.sync_copy(cntv, counts_sh.at[SH_OFF + expert])
    plsc.subcore_barrier()
    pltpu.sync_copy(counts_sh.at[pl.ds(SH_OFF, NUM_EXPERTS)], allcnt)

    base = jnp.int32(0)
    end = jnp.int32(0)
    ends = []
    for e in range(NUM_EXPERTS):
        ce = allcnt[e][0]
        pe = ((ce + BB - 1) >> 8) << 8
        base = base + jnp.where(jnp.int32(e) < expert, pe, 0)
        end = end + pe
        ends.append(end)
    base = pl.multiple_of(base, BB)
    pcnt = ((cnt + BB - 1) >> 8) << 8
    pcnt = pl.multiple_of(pcnt, BB)

    def sbody(j, carry):
        lane = j * L + iota
        tv = tokv[pl.ds(j * L, L)]
        tokv[pl.ds(j * L, L)] = jnp.where(lane < cnt, tv, 0)
        return carry

    lax.fori_loop(cnt >> 4, pcnt >> 4, sbody, 0)

    def pbody(j, carry):
        lane = j * L + iota
        m = lane < cnt
        idxv = prsv[pl.ds(j * L, L)]
        plsc.store_scatter(posv, [idxv], base + lane, mask=m)
        return carry

    lax.fori_loop(0, (cnt + L - 1) >> 4, pbody, 0)

    @pl.when(c == 0)
    def _():
        pltpu.sync_copy(
            posv,
            posp_hbm.at[pl.ds(pl.multiple_of(expert * N_PAIRS, N_PAIRS), N_PAIRS)],
        )

    @pl.when(c == 0)
    def _():
        for ch in range(N_PAIRS // BB):
            @pl.when(ch * BB < pcnt)
            def _(ch=ch):
                pltpu.sync_copy(
                    gtsv.at[pl.ds(ch * BB, BB)],
                    gs_hbm.at[pl.ds(pl.multiple_of(base + ch * BB, BB), BB)],
                )

    @pl.when((c == 0) & (s == 0))
    def _():
        for j in range(NBLK_PAD // L):
            startv = (j * L + iota) * BB
            acc = jnp.zeros((L,), jnp.int32)
            for e in range(NUM_EXPERTS):
                acc = acc + jnp.where(startv >= ends[e], 1, 0)
            blkv[pl.ds(j * L, L)] = jnp.minimum(acc, NUM_EXPERTS - 1)
        pltpu.sync_copy(blkv, blk_hbm)

    half = pcnt >> 1
    lo = c * half

    def gbody(i, carry):
        off = pl.multiple_of(lo + i * 8, 8)
        pltpu.async_copy(x_hbm.at[tokv.at[pl.ds(off, 8)]], xbuf, sem).wait()
        pltpu.sync_copy(xbuf, xg_hbm.at[pl.ds(pl.multiple_of(base + off, 8), 8)])
        return carry

    lax.fori_loop(0, half >> 3, gbody, 0)


def _dispatch(ep, gp, x):
    mesh = plsc.VectorSubcoreMesh(
        core_axis_name="c", subcore_axis_name="s", num_cores=NC, num_subcores=NS
    )
    f = pl.kernel(
        _dispatch_body,
        out_type=[
            jax.ShapeDtypeStruct((P_MAX, D_MODEL), jnp.float32),
            jax.ShapeDtypeStruct((P_MAX,), jnp.float32),
            jax.ShapeDtypeStruct((NBLK_PAD,), jnp.int32),
            jax.ShapeDtypeStruct((NUM_EXPERTS * N_PAIRS,), jnp.int32),
        ],
        mesh=mesh,
        scratch_types=[
            pltpu.VMEM((N_PAIRS,), jnp.int32),
            pltpu.VMEM((N_PAIRS,), jnp.float32),
            pltpu.VMEM((N_PAIRS,), jnp.int32),
            pltpu.VMEM((N_PAIRS,), jnp.float32),
            pltpu.VMEM((N_PAIRS,), jnp.int32),
            pltpu.VMEM((N_PAIRS,), jnp.int32),
            pltpu.VMEM((L,), jnp.int32),
            pltpu.VMEM((NUM_EXPERTS, L), jnp.int32),
            pltpu.VMEM((NBLK_PAD,), jnp.int32),
            pltpu.VMEM((8, D_MODEL), jnp.float32),
            pltpu.VMEM_SHARED((SH_OFF + NUM_EXPERTS, L), jnp.int32),
            pltpu.SemaphoreType.DMA,
        ],
        compiler_params=pltpu.CompilerParams(needs_layout_passes=False),
    )
    return f(ep, gp, x)


def _gmm_body(be_sref, xg_ref, w_ref, b_ref, gs_ref, y_ref):
    y_ref[...] = (
        jnp.dot(xg_ref[...], w_ref[0], preferred_element_type=jnp.float32)
        + b_ref[0]
    ) * gs_ref[...]


def _grouped_mm(blk, xg, gs, W_experts, b_experts):
    grid_spec = pltpu.PrefetchScalarGridSpec(
        num_scalar_prefetch=1,
        grid=(NB,),
        in_specs=[
            pl.BlockSpec((BB, D_MODEL), lambda i, be: (i, 0)),
            pl.BlockSpec((1, D_MODEL, D_MODEL), lambda i, be: (be[i], 0, 0)),
            pl.BlockSpec((1, 1, D_MODEL), lambda i, be: (be[i], 0, 0)),
            pl.BlockSpec((BB, 1), lambda i, be: (i, 0)),
        ],
        out_specs=pl.BlockSpec((BB, D_MODEL), lambda i, be: (i, 0)),
    )
    return pl.pallas_call(
        _gmm_body,
        grid_spec=grid_spec,
        out_shape=jax.ShapeDtypeStruct((P_MAX, D_MODEL), jnp.float32),
        compiler_params=pltpu.CompilerParams(
            dimension_semantics=("arbitrary",),
        ),
    )(
        blk,
        xg,
        W_experts,
        b_experts.reshape(NUM_EXPERTS, 1, D_MODEL),
        gs.reshape(P_MAX, 1),
    )


def _unsort_body(y_hbm, posp_hbm, out_hbm, pp, pidx, ybuf, obuf, sem):
    c = lax.axis_index("c")
    s = lax.axis_index("s")
    wid = s * NC + c
    npair_w = N_PAIRS // (NC * NS)

    for e in range(NUM_EXPERTS):
        pltpu.sync_copy(
            posp_hbm.at[
                pl.ds(pl.multiple_of(e * N_PAIRS + wid * npair_w, npair_w), npair_w)
            ],
            pp.at[pl.ds(e * npair_w, npair_w)],
        )
    for j in range(npair_w // L):
        acc = jnp.zeros((L,), jnp.int32)
        for e in range(NUM_EXPERTS):
            acc = acc + pp[pl.ds(e * npair_w + j * L, L)]
        pidx[pl.ds(j * L, L)] = acc

    def gbody(ch, carry):
        pltpu.async_copy(y_hbm.at[pidx.at[pl.ds(ch * 8, 8)]], ybuf, sem).wait()
        for i in range(4):
            for d in range(D_MODEL // L):
                obuf[i, pl.ds(d * L, L)] = (
                    ybuf[2 * i, pl.ds(d * L, L)] + ybuf[2 * i + 1, pl.ds(d * L, L)]
                )
        pltpu.sync_copy(obuf, out_hbm.at[pl.ds(wid * 128 + ch * 4, 4)])
        return carry

    lax.fori_loop(0, npair_w // 8, gbody, 0)


def _unsort(y, posp):
    mesh = plsc.VectorSubcoreMesh(
        core_axis_name="c", subcore_axis_name="s", num_cores=NC, num_subcores=NS
    )
    f = pl.kernel(
        _unsort_body,
        out_type=jax.ShapeDtypeStruct((N_TOKENS, D_MODEL), jnp.float32),
        mesh=mesh,
        scratch_types=[
            pltpu.VMEM((NUM_EXPERTS * (N_PAIRS // (NC * NS)),), jnp.int32),
            pltpu.VMEM((N_PAIRS // (NC * NS),), jnp.int32),
            pltpu.VMEM((8, D_MODEL), jnp.float32),
            pltpu.VMEM((4, D_MODEL), jnp.float32),
            pltpu.SemaphoreType.DMA,
        ],
        compiler_params=pltpu.CompilerParams(needs_layout_passes=False),
    )
    return f(y, posp)


def kernel(x, W_router, b_router, W_experts, b_experts):
    ei, gg = _router(x, W_router, b_router)
    ep = ei.reshape(N_PAIRS)
    gp = gg.reshape(N_PAIRS)
    xg, gs, blk, posp = _dispatch(ep, gp, x)
    y = _grouped_mm(blk, xg, gs, W_experts, b_experts)
    return _unsort(y, posp)


# speedup vs baseline: 1.2711x; 1.0697x over previous
import functools

import jax
import jax.numpy as jnp
from jax import lax
from jax.experimental import pallas as pl
from jax.experimental.pallas import tpu as pltpu
from jax.experimental.pallas import tpu_sc as plsc

NUM_EXPERTS = 16
TOP_K = 2
N_TOKENS = 4096
D_MODEL = 1024
N_PAIRS = N_TOKENS * TOP_K
BB = 256
P_MAX = N_PAIRS + NUM_EXPERTS * BB
NB = P_MAX // BB
NBLK_PAD = 64
NC, NS, L = 2, 16, 16
SH_OFF = 32
GCH = 32
UCH = 16


def _router_body(x_ref, wr_ref, br_ref, ei_ref, g_ref):
    logits = (
        jnp.dot(x_ref[...], wr_ref[...], preferred_element_type=jnp.float32)
        + br_ref[...]
    )
    iota = jax.lax.broadcasted_iota(jnp.int32, logits.shape, 1)
    m1 = jnp.max(logits, axis=-1, keepdims=True)
    e0 = jnp.min(
        jnp.where(logits >= m1, iota, NUM_EXPERTS), axis=-1, keepdims=True
    )
    masked = jnp.where(logits >= m1, -jnp.inf, logits)
    m2 = jnp.max(masked, axis=-1, keepdims=True)
    e1 = jnp.min(
        jnp.where(masked >= m2, iota, NUM_EXPERTS), axis=-1, keepdims=True
    )
    t = jnp.exp(m2 - m1)
    g0 = 1.0 / (1.0 + t)
    ei_ref[...] = jnp.concatenate([e0, e1], axis=1)
    g_ref[...] = jnp.concatenate([g0, 1.0 - g0], axis=1)


def _router(x, W_router, b_router):
    bt = 1024
    return pl.pallas_call(
        _router_body,
        grid=(N_TOKENS // bt,),
        in_specs=[
            pl.BlockSpec((bt, D_MODEL), lambda i: (i, 0)),
            pl.BlockSpec((D_MODEL, NUM_EXPERTS), lambda i: (0, 0)),
            pl.BlockSpec((1, NUM_EXPERTS), lambda i: (0, 0)),
        ],
        out_specs=[
            pl.BlockSpec((bt, TOP_K), lambda i: (i, 0)),
            pl.BlockSpec((bt, TOP_K), lambda i: (i, 0)),
        ],
        out_shape=[
            jax.ShapeDtypeStruct((N_TOKENS, TOP_K), jnp.int32),
            jax.ShapeDtypeStruct((N_TOKENS, TOP_K), jnp.float32),
        ],
    )(x, W_router, b_router.reshape(1, NUM_EXPERTS))


def _dispatch_body(
    ep_hbm, gp_hbm, x_hbm,
    xg_hbm, gs_hbm, blk_hbm, posp_hbm,
    epv, gv, tokv, gtsv, prsv, posv, cntv, allcnt, blkv,
    xbuf0, xbuf1, counts_sh, sem0, sem1,
):
    c = lax.axis_index("c")
    s = lax.axis_index("s")
    expert = s
    iota = lax.iota(jnp.int32, L)

    pltpu.sync_copy(ep_hbm, epv)
    pltpu.sync_copy(gp_hbm, gv)

    def zbody(j, carry):
        posv[pl.ds(j * L, L)] = jnp.zeros((L,), jnp.int32)
        return carry

    lax.fori_loop(0, N_PAIRS // L, zbody, 0)

    def cbody(j, cnt):
        tot = cnt
        for u in range(4):
            k = j * 4 + u
            ev = epv[pl.ds(k * L, L)]
            mask = ev == expert
            pvec = k * L + iota
            cs = plsc.cumsum(jnp.where(mask, 1, 0))
            dest = tot + cs - 1
            plsc.store_scatter(tokv, [dest], pvec >> 1, mask=mask)
            plsc.store_scatter(gtsv, [dest], gv[pl.ds(k * L, L)], mask=mask)
            plsc.store_scatter(prsv, [dest], pvec, mask=mask)
            tot = tot + cs[L - 1]
        return tot

    cnt = lax.fori_loop(0, N_PAIRS // (4 * L), cbody, jnp.int32(0))

    cntv[...] = jnp.broadcast_to(cnt, (L,))
    pltpu.sync_copy(cntv, counts_sh.at[SH_OFF + expert])
    plsc.subcore_barrier()
    pltpu.sync_copy(counts_sh.at[pl.ds(SH_OFF, NUM_EXPERTS)], allcnt)

    base = jnp.int32(0)
    end = jnp.int32(0)
    ends = []
    for e in range(NUM_EXPERTS):
        ce = allcnt[e][0]
        pe = ((ce + BB - 1) >> 8) << 8
        base = base + jnp.where(jnp.int32(e) < expert, pe, 0)
        end = end + pe
        ends.append(end)
    base = pl.multiple_of(base, BB)
    pcnt = ((cnt + BB - 1) >> 8) << 8
    pcnt = pl.multiple_of(pcnt, BB)

    def sbody(j, carry):
        lane = j * L + iota
        tv = tokv[pl.ds(j * L, L)]
        tokv[pl.ds(j * L, L)] = jnp.where(lane < cnt, tv, 0)
        return carry

    lax.fori_loop(cnt >> 4, pcnt >> 4, sbody, 0)

    def pbody(j, carry):
        lane = j * L + iota
        m = lane < cnt
        idxv = prsv[pl.ds(j * L, L)]
        plsc.store_scatter(posv, [idxv], base + lane, mask=m)
        return carry

    lax.fori_loop(0, (cnt + L - 1) >> 4, pbody, 0)

    @pl.when(c == 0)
    def _():
        pltpu.sync_copy(
            posv,
            posp_hbm.at[pl.ds(pl.multiple_of(expert * N_PAIRS, N_PAIRS), N_PAIRS)],
        )

    @pl.when(c == 0)
    def _():
        for ch in range(N_PAIRS // BB):
            @pl.when(ch * BB < pcnt)
            def _(ch=ch):
                pltpu.sync_copy(
                    gtsv.at[pl.ds(ch * BB, BB)],
                    gs_hbm.at[pl.ds(pl.multiple_of(base + ch * BB, BB), BB)],
                )

    @pl.when((c == 0) & (s == 0))
    def _():
        for j in range(NBLK_PAD // L):
            startv = (j * L + iota) * BB
            acc = jnp.zeros((L,), jnp.int32)
            for e in range(NUM_EXPERTS):
                acc = acc + jnp.where(startv >= ends[e], 1, 0)
            blkv[pl.ds(j * L, L)] = jnp.minimum(acc, NUM_EXPERTS - 1)
        pltpu.sync_copy(blkv, blk_hbm)

    half = pcnt >> 1
    lo = c * half
    nch = half // GCH
    bufs = (xbuf0, xbuf1)
    sems = (sem0, sem1)

    def _fire(ci, b):
        off = pl.multiple_of(lo + ci * GCH, 8)
        pltpu.async_copy(x_hbm.at[tokv.at[pl.ds(off, GCH)]], bufs[b], sems[b])

    def _wait(b):
        pltpu.make_async_copy(
            x_hbm.at[pl.ds(0, GCH)], bufs[b], sems[b]
        ).wait()

    def _flush(ci, b):
        off = pl.multiple_of(base + lo + ci * GCH, 8)
        pltpu.sync_copy(bufs[b], xg_hbm.at[pl.ds(off, GCH)])

    @pl.when(nch > 0)
    def _():
        _fire(0, 0)

    def gbody(j, carry):
        c0 = 2 * j
        c1 = 2 * j + 1

        @pl.when(c1 < nch)
        def _():
            _fire(c1, 1)

        @pl.when(c0 < nch)
        def _():
            _wait(0)
            _flush(c0, 0)

        @pl.when(c1 + 1 < nch)
        def _():
            _fire(c1 + 1, 0)

        @pl.when(c1 < nch)
        def _():
            _wait(1)
            _flush(c1, 1)

        return carry

    lax.fori_loop(0, (nch + 1) >> 1, gbody, 0)


def _dispatch(ep, gp, x):
    mesh = plsc.VectorSubcoreMesh(
        core_axis_name="c", subcore_axis_name="s", num_cores=NC, num_subcores=NS
    )
    f = pl.kernel(
        _dispatch_body,
        out_type=[
            jax.ShapeDtypeStruct((P_MAX, D_MODEL), jnp.float32),
            jax.ShapeDtypeStruct((P_MAX,), jnp.float32),
            jax.ShapeDtypeStruct((NBLK_PAD,), jnp.int32),
            jax.ShapeDtypeStruct((NUM_EXPERTS * N_PAIRS,), jnp.int32),
        ],
        mesh=mesh,
        scratch_types=[
            pltpu.VMEM((N_PAIRS,), jnp.int32),
            pltpu.VMEM((N_PAIRS,), jnp.float32),
            pltpu.VMEM((N_PAIRS,), jnp.int32),
            pltpu.VMEM((N_PAIRS,), jnp.float32),
            pltpu.VMEM((N_PAIRS,), jnp.int32),
            pltpu.VMEM((N_PAIRS,), jnp.int32),
            pltpu.VMEM((L,), jnp.int32),
            pltpu.VMEM((NUM_EXPERTS, L), jnp.int32),
            pltpu.VMEM((NBLK_PAD,), jnp.int32),
            pltpu.VMEM((GCH, D_MODEL), jnp.float32),
            pltpu.VMEM((GCH, D_MODEL), jnp.float32),
            pltpu.VMEM_SHARED((SH_OFF + NUM_EXPERTS, L), jnp.int32),
            pltpu.SemaphoreType.DMA,
            pltpu.SemaphoreType.DMA,
        ],
        compiler_params=pltpu.CompilerParams(needs_layout_passes=False),
    )
    return f(ep, gp, x)


def _gmm_body(be_sref, xg_ref, w_ref, b_ref, gs_ref, y_ref):
    y_ref[...] = (
        jnp.dot(xg_ref[...], w_ref[0], preferred_element_type=jnp.float32)
        + b_ref[0]
    ) * gs_ref[...]


def _grouped_mm(blk, xg, gs, W_experts, b_experts):
    grid_spec = pltpu.PrefetchScalarGridSpec(
        num_scalar_prefetch=1,
        grid=(NB,),
        in_specs=[
            pl.BlockSpec((BB, D_MODEL), lambda i, be: (i, 0)),
            pl.BlockSpec((1, D_MODEL, D_MODEL), lambda i, be: (be[i], 0, 0)),
            pl.BlockSpec((1, 1, D_MODEL), lambda i, be: (be[i], 0, 0)),
            pl.BlockSpec((BB, 1), lambda i, be: (i, 0)),
        ],
        out_specs=pl.BlockSpec((BB, D_MODEL), lambda i, be: (i, 0)),
    )
    return pl.pallas_call(
        _gmm_body,
        grid_spec=grid_spec,
        out_shape=jax.ShapeDtypeStruct((P_MAX, D_MODEL), jnp.float32),
        compiler_params=pltpu.CompilerParams(
            dimension_semantics=("arbitrary",),
        ),
    )(
        blk,
        xg,
        W_experts,
        b_experts.reshape(NUM_EXPERTS, 1, D_MODEL),
        gs.reshape(P_MAX, 1),
    )


def _unsort_body(
    y_hbm, posp_hbm, out_hbm, pp, pidx, ybuf0, ybuf1, obuf0, obuf1, sem0, sem1
):
    c = lax.axis_index("c")
    s = lax.axis_index("s")
    wid = s * NC + c
    npair_w = N_PAIRS // (NC * NS)

    for e in range(NUM_EXPERTS):
        pltpu.sync_copy(
            posp_hbm.at[
                pl.ds(pl.multiple_of(e * N_PAIRS + wid * npair_w, npair_w), npair_w)
            ],
            pp.at[pl.ds(e * npair_w, npair_w)],
        )
    for j in range(npair_w // L):
        acc = jnp.zeros((L,), jnp.int32)
        for e in range(NUM_EXPERTS):
            acc = acc + pp[pl.ds(e * npair_w + j * L, L)]
        pidx[pl.ds(j * L, L)] = acc

    nch = npair_w // UCH
    ybufs = (ybuf0, ybuf1)
    obufs = (obuf0, obuf1)
    sems = (sem0, sem1)

    def _fire(ci, b):
        pltpu.async_copy(
            y_hbm.at[pidx.at[pl.ds(ci * UCH, UCH)]], ybufs[b], sems[b]
        )

    def _wait(b):
        pltpu.make_async_copy(y_hbm.at[pl.ds(0, UCH)], ybufs[b], sems[b]).wait()

    def _combine_flush(ci, b):
        yb = ybufs[b]
        ob = obufs[b]
        for i in range(UCH // 2):
            for d in range(D_MODEL // L):
                ob[i, pl.ds(d * L, L)] = (
                    yb[2 * i, pl.ds(d * L, L)] + yb[2 * i + 1, pl.ds(d * L, L)]
                )
        pltpu.sync_copy(
            ob, out_hbm.at[pl.ds(wid * (npair_w // 2) + ci * (UCH // 2), UCH // 2)]
        )

    _fire(0, 0)

    def gbody(j, carry):
        c0 = 2 * j
        c1 = 2 * j + 1
        _fire(c1, 1)
        _wait(0)
        _combine_flush(c0, 0)

        @pl.when(c1 + 1 < nch)
        def _():
            _fire(c1 + 1, 0)

        _wait(1)
        _combine_flush(c1, 1)
        return carry

    lax.fori_loop(0, nch // 2, gbody, 0)


def _unsort(y, posp):
    mesh = plsc.VectorSubcoreMesh(
        core_axis_name="c", subcore_axis_name="s", num_cores=NC, num_subcores=NS
    )
    f = pl.kernel(
        _unsort_body,
        out_type=jax.ShapeDtypeStruct((N_TOKENS, D_MODEL), jnp.float32),
        mesh=mesh,
        scratch_types=[
            pltpu.VMEM((NUM_EXPERTS * (N_PAIRS // (NC * NS)),), jnp.int32),
            pltpu.VMEM((N_PAIRS // (NC * NS),), jnp.int32),
            pltpu.VMEM((UCH, D_MODEL), jnp.float32),
            pltpu.VMEM((UCH, D_MODEL), jnp.float32),
            pltpu.VMEM((UCH // 2, D_MODEL), jnp.float32),
            pltpu.VMEM((UCH // 2, D_MODEL), jnp.float32),
            pltpu.SemaphoreType.DMA,
            pltpu.SemaphoreType.DMA,
        ],
        compiler_params=pltpu.CompilerParams(needs_layout_passes=False),
    )
    return f(y, posp)


def kernel(x, W_router, b_router, W_experts, b_experts):
    ei, gg = _router(x, W_router, b_router)
    ep = ei.reshape(N_PAIRS)
    gp = gg.reshape(N_PAIRS)
    xg, gs, blk, posp = _dispatch(ep, gp, x)
    y = _grouped_mm(blk, xg, gs, W_experts, b_experts)
    return _unsort(y, posp)


# speedup vs baseline: 2.3595x; 1.8563x over previous
import functools

import jax
import jax.numpy as jnp
from jax import lax
from jax.experimental import pallas as pl
from jax.experimental.pallas import tpu as pltpu
from jax.experimental.pallas import tpu_sc as plsc

NUM_EXPERTS = 16
TOP_K = 2
N_TOKENS = 4096
D_MODEL = 1024
N_PAIRS = N_TOKENS * TOP_K
BB = 256
P_MAX = N_PAIRS + NUM_EXPERTS * BB
NB = P_MAX // BB
NBLK_PAD = 64
NC, NS, L = 2, 16, 16
BT = 1024
NBT = N_TOKENS // BT
TW = N_TOKENS // (NC * NS)
TCH = 16
NCH = TW // TCH


def _router_body(x_ref, wr_ref, br_ref, ei_ref, g_ref, hcnt_ref):
    logits = (
        jnp.dot(x_ref[...], wr_ref[...], preferred_element_type=jnp.float32)
        + br_ref[...]
    )
    iota = jax.lax.broadcasted_iota(jnp.int32, logits.shape, 1)
    m1 = jnp.max(logits, axis=-1, keepdims=True)
    e0 = jnp.min(
        jnp.where(logits >= m1, iota, NUM_EXPERTS), axis=-1, keepdims=True
    )
    masked = jnp.where(logits >= m1, -jnp.inf, logits)
    m2 = jnp.max(masked, axis=-1, keepdims=True)
    e1 = jnp.min(
        jnp.where(masked >= m2, iota, NUM_EXPERTS), axis=-1, keepdims=True
    )
    t = jnp.exp(m2 - m1)
    g0 = 1.0 / (1.0 + t)
    ei_ref[...] = jnp.concatenate([e0, e1], axis=1)
    g_ref[...] = jnp.concatenate([g0, 1.0 - g0], axis=1)
    h = (iota == e0).astype(jnp.float32) + (iota == e1).astype(jnp.float32)
    hcnt_ref[...] = jnp.sum(h, axis=0, keepdims=True).reshape(1, 1, NUM_EXPERTS)


def _router(x, W_router, b_router):
    return pl.pallas_call(
        _router_body,
        grid=(NBT,),
        in_specs=[
            pl.BlockSpec((BT, D_MODEL), lambda i: (i, 0)),
            pl.BlockSpec((D_MODEL, NUM_EXPERTS), lambda i: (0, 0)),
            pl.BlockSpec((1, NUM_EXPERTS), lambda i: (0, 0)),
        ],
        out_specs=[
            pl.BlockSpec((BT, TOP_K), lambda i: (i, 0)),
            pl.BlockSpec((BT, TOP_K), lambda i: (i, 0)),
            pl.BlockSpec((1, 1, NUM_EXPERTS), lambda i: (i, 0, 0)),
        ],
        out_shape=[
            jax.ShapeDtypeStruct((N_TOKENS, TOP_K), jnp.int32),
            jax.ShapeDtypeStruct((N_TOKENS, TOP_K), jnp.float32),
            jax.ShapeDtypeStruct((NBT, 1, NUM_EXPERTS), jnp.float32),
        ],
    )(x, W_router, b_router.reshape(1, NUM_EXPERTS))


def _plan_body(ei_ref, hcnt_ref, pos_ref, blk_ref, carry_scr):
    i = pl.program_id(0)

    @pl.when(i == 0)
    def _():
        carry_scr[...] = jnp.zeros_like(carry_scr)

    e0 = ei_ref[:, 0:1]
    e1 = ei_ref[:, 1:2]
    io = jax.lax.broadcasted_iota(jnp.int32, (BT, NUM_EXPERTS), 1)
    oh0 = (io == e0).astype(jnp.float32)
    oh1 = (io == e1).astype(jnp.float32)
    h = oh0 + oh1

    counts = jnp.sum(hcnt_ref[...], axis=0)
    pe = jnp.ceil(counts * (1.0 / BB)) * BB
    r16 = jax.lax.broadcasted_iota(jnp.int32, (NUM_EXPERTS, NUM_EXPERTS), 0)
    c16 = jax.lax.broadcasted_iota(jnp.int32, (NUM_EXPERTS, NUM_EXPERTS), 1)
    tri16 = (r16 <= c16).astype(jnp.float32)
    ends = jnp.dot(pe, tri16, preferred_element_type=jnp.float32)
    base = ends - pe

    rr = jax.lax.broadcasted_iota(jnp.int32, (BT, BT), 0)
    cc = jax.lax.broadcasted_iota(jnp.int32, (BT, BT), 1)
    tstrict = (cc < rr).astype(jnp.float32)
    ex = jnp.dot(tstrict, h, preferred_element_type=jnp.float32) + carry_scr[...]

    rank0 = jnp.sum(ex * oh0, axis=1, keepdims=True)
    rank1 = jnp.sum(ex * oh1, axis=1, keepdims=True)
    base0 = jnp.sum(base * oh0, axis=1, keepdims=True)
    base1 = jnp.sum(base * oh1, axis=1, keepdims=True)
    p0 = (base0 + rank0).astype(jnp.int32)
    p1 = (base1 + rank1).astype(jnp.int32)
    pos_ref[...] = jnp.concatenate([p0, p1], axis=1)

    carry_scr[...] += jnp.sum(h, axis=0, keepdims=True)

    @pl.when(i == NBT - 1)
    def _():
        bstart = (
            jax.lax.broadcasted_iota(jnp.int32, (NBLK_PAD, NUM_EXPERTS), 0) * BB
        ).astype(jnp.float32)
        acc = jnp.sum((bstart >= ends).astype(jnp.float32), axis=1, keepdims=True)
        blk_ref[...] = jnp.minimum(acc, NUM_EXPERTS - 1).astype(jnp.int32)


def _plan(ei, hcnt):
    return pl.pallas_call(
        _plan_body,
        grid=(NBT,),
        in_specs=[
            pl.BlockSpec((BT, TOP_K), lambda i: (i, 0)),
            pl.BlockSpec((NBT, 1, NUM_EXPERTS), lambda i: (0, 0, 0)),
        ],
        out_specs=[
            pl.BlockSpec((BT, TOP_K), lambda i: (i, 0)),
            pl.BlockSpec((NBLK_PAD, 1), lambda i: (0, 0)),
        ],
        out_shape=[
            jax.ShapeDtypeStruct((N_TOKENS, TOP_K), jnp.int32),
            jax.ShapeDtypeStruct((NBLK_PAD, 1), jnp.int32),
        ],
        scratch_shapes=[pltpu.VMEM((1, NUM_EXPERTS), jnp.float32)],
        compiler_params=pltpu.CompilerParams(
            dimension_semantics=("arbitrary",),
        ),
    )(ei, hcnt)


def _scatter_body(
    x_hbm, pos0_hbm, pos1_hbm, xg_hbm,
    p0v, p1v, xb0, xb1, semA0, semB0, semA1, semB1,
):
    c = lax.axis_index("c")
    s = lax.axis_index("s")
    wid = s * NC + c
    row0 = pl.multiple_of(wid * (TW // TCH), 8)
    pltpu.sync_copy(pos0_hbm.at[pl.ds(row0, NCH)], p0v)
    pltpu.sync_copy(pos1_hbm.at[pl.ds(row0, NCH)], p1v)

    xbufs = (xb0, xb1)
    semsA = (semA0, semA1)
    semsB = (semB0, semB1)

    for ch in range(NCH):
        b = ch & 1
        if ch >= 2:
            pltpu.make_async_copy(
                xbufs[b], xg_hbm.at[p0v.at[ch - 2]], semsA[b]
            ).wait()
            pltpu.make_async_copy(
                xbufs[b], xg_hbm.at[p1v.at[ch - 2]], semsB[b]
            ).wait()
        tok = pl.multiple_of(wid * TW + ch * TCH, 8)
        pltpu.sync_copy(x_hbm.at[pl.ds(tok, TCH)], xbufs[b])
        pltpu.async_copy(xbufs[b], xg_hbm.at[p0v.at[ch]], semsA[b])
        pltpu.async_copy(xbufs[b], xg_hbm.at[p1v.at[ch]], semsB[b])
    for ch in (NCH - 2, NCH - 1):
        b = ch & 1
        pltpu.make_async_copy(xbufs[b], xg_hbm.at[p0v.at[ch]], semsA[b]).wait()
        pltpu.make_async_copy(xbufs[b], xg_hbm.at[p1v.at[ch]], semsB[b]).wait()


def _scatter(x, pos0, pos1):
    mesh = plsc.VectorSubcoreMesh(
        core_axis_name="c", subcore_axis_name="s", num_cores=NC, num_subcores=NS
    )
    f = pl.kernel(
        _scatter_body,
        out_type=jax.ShapeDtypeStruct((P_MAX, D_MODEL), jnp.float32),
        mesh=mesh,
        scratch_types=[
            pltpu.VMEM((NCH, TCH), jnp.int32),
            pltpu.VMEM((NCH, TCH), jnp.int32),
            pltpu.VMEM((TCH, D_MODEL), jnp.float32),
            pltpu.VMEM((TCH, D_MODEL), jnp.float32),
            pltpu.SemaphoreType.DMA,
            pltpu.SemaphoreType.DMA,
            pltpu.SemaphoreType.DMA,
            pltpu.SemaphoreType.DMA,
        ],
        compiler_params=pltpu.CompilerParams(needs_layout_passes=False),
    )
    return f(x, pos0, pos1)


def _gmm_body(be_sref, xg_ref, w_ref, b_ref, y_ref):
    y_ref[...] = (
        jnp.dot(xg_ref[...], w_ref[0], preferred_element_type=jnp.float32)
        + b_ref[0]
    )


def _grouped_mm(blk, xg, W_experts, b_experts):
    grid_spec = pltpu.PrefetchScalarGridSpec(
        num_scalar_prefetch=1,
        grid=(NB,),
        in_specs=[
            pl.BlockSpec((BB, D_MODEL), lambda i, be: (i, 0)),
            pl.BlockSpec((1, D_MODEL, D_MODEL), lambda i, be: (be[i], 0, 0)),
            pl.BlockSpec((1, 1, D_MODEL), lambda i, be: (be[i], 0, 0)),
        ],
        out_specs=pl.BlockSpec((BB, D_MODEL), lambda i, be: (i, 0)),
    )
    return pl.pallas_call(
        _gmm_body,
        grid_spec=grid_spec,
        out_shape=jax.ShapeDtypeStruct((P_MAX, D_MODEL), jnp.float32),
        compiler_params=pltpu.CompilerParams(
            dimension_semantics=("arbitrary",),
        ),
    )(
        blk,
        xg,
        W_experts,
        b_experts.reshape(NUM_EXPERTS, 1, D_MODEL),
    )


def _unsort_body(
    y_hbm, pos0_hbm, pos1_hbm, g0_hbm, g1_hbm, out_hbm,
    p0v, p1v, g0v, g1v, ya0, ya1, yb0, yb1, ob0, ob1,
    semA0, semB0, semA1, semB1,
):
    c = lax.axis_index("c")
    s = lax.axis_index("s")
    wid = s * NC + c
    row0 = pl.multiple_of(wid * NCH, 8)
    tok0 = pl.multiple_of(wid * TW, 8)
    pltpu.sync_copy(pos0_hbm.at[pl.ds(row0, NCH)], p0v)
    pltpu.sync_copy(pos1_hbm.at[pl.ds(row0, NCH)], p1v)
    pltpu.sync_copy(g0_hbm.at[pl.ds(tok0, TW)], g0v)
    pltpu.sync_copy(g1_hbm.at[pl.ds(tok0, TW)], g1v)

    yas = (ya0, ya1)
    ybs = (yb0, yb1)
    obs = (ob0, ob1)
    semsA = (semA0, semA1)
    semsB = (semB0, semB1)

    def _fire(ch, b):
        pltpu.async_copy(y_hbm.at[p0v.at[ch]], yas[b], semsA[b])
        pltpu.async_copy(y_hbm.at[p1v.at[ch]], ybs[b], semsB[b])

    def _wait(b):
        pltpu.make_async_copy(y_hbm.at[pl.ds(0, TCH)], yas[b], semsA[b]).wait()
        pltpu.make_async_copy(y_hbm.at[pl.ds(0, TCH)], ybs[b], semsB[b]).wait()

    def _combine(ch, b):
        ga = g0v[pl.ds(ch * TCH, TCH)]
        gb = g1v[pl.ds(ch * TCH, TCH)]

        def dbody(d, carry):
            for i in range(TCH):
                obs[b][i, pl.ds(d * L, L)] = (
                    yas[b][i, pl.ds(d * L, L)] * ga[i]
                    + ybs[b][i, pl.ds(d * L, L)] * gb[i]
                )
            return carry

        lax.fori_loop(0, D_MODEL // L, dbody, 0)
        pltpu.sync_copy(
            obs[b], out_hbm.at[pl.ds(pl.multiple_of(tok0 + ch * TCH, 8), TCH)]
        )

    _fire(0, 0)
    for ch in range(NCH):
        b = ch & 1
        if ch + 1 < NCH:
            _fire(ch + 1, 1 - b)
        _wait(b)
        _combine(ch, b)


def _unsort(y, pos0, pos1, g0, g1):
    mesh = plsc.VectorSubcoreMesh(
        core_axis_name="c", subcore_axis_name="s", num_cores=NC, num_subcores=NS
    )
    f = pl.kernel(
        _unsort_body,
        out_type=jax.ShapeDtypeStruct((N_TOKENS, D_MODEL), jnp.float32),
        mesh=mesh,
        scratch_types=[
            pltpu.VMEM((NCH, TCH), jnp.int32),
            pltpu.VMEM((NCH, TCH), jnp.int32),
            pltpu.VMEM((TW,), jnp.float32),
            pltpu.VMEM((TW,), jnp.float32),
            pltpu.VMEM((TCH, D_MODEL), jnp.float32),
            pltpu.VMEM((TCH, D_MODEL), jnp.float32),
            pltpu.VMEM((TCH, D_MODEL), jnp.float32),
            pltpu.VMEM((TCH, D_MODEL), jnp.float32),
            pltpu.VMEM((TCH, D_MODEL), jnp.float32),
            pltpu.VMEM((TCH, D_MODEL), jnp.float32),
            pltpu.SemaphoreType.DMA,
            pltpu.SemaphoreType.DMA,
            pltpu.SemaphoreType.DMA,
            pltpu.SemaphoreType.DMA,
        ],
        compiler_params=pltpu.CompilerParams(needs_layout_passes=False),
    )
    return f(y, pos0, pos1, g0, g1)


def kernel(x, W_router, b_router, W_experts, b_experts):
    ei, gg, hcnt = _router(x, W_router, b_router)
    pos, blk = _plan(ei, hcnt)
    pos0 = pos[:, 0].reshape(N_TOKENS // TCH, TCH)
    pos1 = pos[:, 1].reshape(N_TOKENS // TCH, TCH)
    xg = _scatter(x, pos0, pos1)
    y = _grouped_mm(blk.reshape(NBLK_PAD), xg, W_experts, b_experts)
    return _unsort(y, pos0, pos1, gg[:, 0], gg[:, 1])
